# Initial kernel scaffold; baseline (speedup 1.0000x reference)
import jax
import jax.numpy as jnp
from jax.experimental import pallas as pl


def kernel(all_embed, weight, kg_val, ii_val, ui_val, users, pos_items, neg_items, kg_pairs, kg_row, ii_src, ii_dst, ui_user, ui_item):
    raise NotImplementedError("write your pallas kernel here")


# speedup vs baseline: 1.0040x; 1.0040x over previous
import functools

import jax
import jax.numpy as jnp
from jax.experimental import pallas as pl
from jax.experimental.pallas import tpu as pltpu

N_USERS = 50000
N_ITEMS = 30000
N_ENTITIES = 80000
N_NODES = 130000
N_REL = 16
DIM = 64
HOPS = 2
BATCH = 4096
DECAY = 1e-4
SCALE = 10.0
ALPHA = 0.1


def _fnorm(x, eps=1e-12):
    n = jnp.sqrt(jnp.sum(x * x, axis=1, keepdims=True))
    return x / jnp.maximum(n, eps)


_BM = 512


def _loss_body(u_kg_blk, u_ii_blk, pos_blk, neg_blk, in_all,
               ssl_ref, mf_ref, reg_ref):
    i = pl.program_id(0)

    u_kg = u_kg_blk[...]
    u_ii = u_ii_blk[...]
    u_e = u_kg + u_ii
    pos_e = pos_blk[...]
    neg_e = neg_blk[...]

    def l2n(x):
        return x / jnp.sqrt(jnp.sum(x * x, -1, keepdims=True) + 1e-24)

    inn = in_all[...]

    kn_blk = l2n(u_kg)
    scores = jnp.dot(kn_blk, inn.T, preferred_element_type=jnp.float32) * SCALE
    m = jnp.max(scores, axis=1, keepdims=True)
    lse = jnp.log(jnp.sum(jnp.exp(scores - m), axis=1, keepdims=True)) + m
    in_blk = l2n(u_ii)
    diag = jnp.sum(kn_blk * in_blk, axis=1, keepdims=True) * SCALE
    ssl_part = jnp.sum(lse - diag)

    d = jnp.sum(u_e * (pos_e - neg_e), axis=1, keepdims=True)
    mf_part = jnp.sum(jnp.maximum(-d, 0.0) +
                      jnp.log1p(jnp.exp(-jnp.abs(d))))

    reg_part = (jnp.sum(u_e * u_e) + jnp.sum(pos_e * pos_e) +
                jnp.sum(neg_e * neg_e))

    @pl.when(i == 0)
    def _init():
        ssl_ref[0, 0] = 0.0
        mf_ref[0, 0] = 0.0
        reg_ref[0, 0] = 0.0

    ssl_ref[0, 0] += ssl_part
    mf_ref[0, 0] += mf_part
    reg_ref[0, 0] += reg_part


def _loss_head(u_kg_b, u_ii_b, pos_e, neg_e):
    def l2n(x):
        return x / jnp.sqrt(jnp.sum(x * x, -1, keepdims=True) + 1e-24)

    inn = l2n(u_ii_b)
    grid = (BATCH // _BM,)
    blk = pl.BlockSpec((_BM, DIM), lambda i: (i, 0))
    full = pl.BlockSpec((BATCH, DIM), lambda i: (0, 0))
    out_spec = pl.BlockSpec(memory_space=pltpu.SMEM)
    ssl_s, mf_s, reg_s = pl.pallas_call(
        _loss_body,
        grid=grid,
        in_specs=[blk, blk, blk, blk, full],
        out_specs=[out_spec, out_spec, out_spec],
        out_shape=[jax.ShapeDtypeStruct((1, 1), jnp.float32)] * 3,
    )(u_kg_b, u_ii_b, pos_e, neg_e, inn)
    ssl_loss = ssl_s[0, 0] / BATCH
    mf_loss = mf_s[0, 0] / BATCH
    emb_loss = DECAY * (reg_s[0, 0] / 2.0) / BATCH
    total = mf_loss + emb_loss + ALPHA * ssl_loss
    return total, mf_loss, emb_loss


def kernel(all_embed, weight, kg_val, ii_val, ui_val, users, pos_items,
           neg_items, kg_pairs, kg_row, ii_src, ii_dst, ui_user, ui_item):
    user_emb = all_embed[:N_USERS]
    entity_emb = all_embed[N_USERS:]
    item_emb = entity_emb[:N_ITEMS]

    kg_rel = kg_pairs[:, 0]
    kg_tail = kg_pairs[:, 1]
    w_edge_scale = kg_val[:, None]

    un1 = entity_emb[kg_tail] * weight[kg_rel - 1] * w_edge_scale
    e1 = _fnorm(jax.ops.segment_sum(un1, kg_row, num_segments=N_ENTITIES))

    un2 = e1[kg_tail] * weight[kg_rel - 1] * w_edge_scale
    e2 = _fnorm(jax.ops.segment_sum(un2, kg_row, num_segments=N_ENTITIES))

    ii_agg = _fnorm(jax.ops.segment_sum(
        item_emb[ii_src] * ii_val[:, None], ii_dst, num_segments=N_ITEMS))

    item_gcn = (2.0 * item_emb + e1[:N_ITEMS] + e2[:N_ITEMS] + 2.0 * ii_agg)

    def ui_mat(x):
        return jax.ops.segment_sum(x[ui_item] * ui_val[:, None], ui_user,
                                   num_segments=N_USERS)

    uiA = _fnorm(ui_mat(item_emb))
    uiB = _fnorm(ui_mat(e1[:N_ITEMS]))
    uiC = _fnorm(ui_mat(ii_agg))

    user_kg = user_emb + uiA + uiB
    user_ii = user_emb + uiA + uiC

    u_kg_b = user_kg[users]
    u_ii_b = user_ii[users]
    pos_e = item_gcn[pos_items]
    neg_e = item_gcn[neg_items]

    return _loss_head(u_kg_b, u_ii_b, pos_e, neg_e)


# speedup vs baseline: 3.9558x; 3.9401x over previous
import functools

import jax
import jax.numpy as jnp
from jax import lax
from jax.experimental import pallas as pl
from jax.experimental.pallas import tpu as pltpu
from jax.experimental.pallas import tpu_sc as plsc

N_USERS = 50000
N_ITEMS = 30000
N_ENTITIES = 80000
N_REL = 16
DIM = 64
BATCH = 4096
DECAY = 1e-4
SCALE = 10.0
ALPHA = 0.1

NC = 2
NS = 16
NW = NC * NS

_C = 512
_R = 1024


def _ceil_div(a, b):
    return (a + b - 1) // b


def _seg_sum_body(nb, use_w, table_h, idx_h, meta_h, boff_h,
                  w_h, out_h, acc, idxv, rows, smeta, bnds, wtab, spm, sem):
    cid = lax.axis_index("c")
    sid = lax.axis_index("s")
    wid = sid * NC + cid

    if use_w:
        pltpu.sync_copy(w_h, wtab)

    zero16 = jnp.zeros((16,), jnp.float32)

    def block_body(bi, _):
        b = wid + bi * NW

        @pl.when(b < nb)
        def _do_block():
            base = b * _R
            pltpu.sync_copy(boff_h.at[b], bnds)
            bv = bnds[...]
            e0 = bv[0]
            e1 = bv[1]

            def zbody(i, _):
                for u in range(4):
                    for c in range(DIM // 16):
                        acc[i * 4 + u, pl.ds(c * 16, 16)] = zero16
                return 0

            lax.fori_loop(0, _R // 4, zbody, 0)

            ealign = (e0 // 128) * 128
            nch = lax.max((e1 - ealign + _C - 1) // _C, 0)

            def chunk_body(k, _):
                estart = ealign + k * _C
                pltpu.sync_copy(meta_h.at[pl.ds(estart * 2, _C * 2)], spm.at[sid])
                pltpu.sync_copy(spm.at[sid], smeta)
                pltpu.sync_copy(idx_h.at[pl.ds(estart, _C)], idxv)
                descs = [
                    pltpu.async_copy(
                        table_h.at[idxv.at[pl.ds(j * 128, 128)]],
                        rows.at[pl.ds(j * 128, 128)], sem)
                    for j in range(_C // 128)
                ]
                for d in descs:
                    d.wait()

                i_lo = lax.max(e0 - estart, 0)
                i_hi = lax.min(e1 - estart, _C)

                def scaled_row(i, c, v):
                    row = rows[i, pl.ds(c * 16, 16)]
                    if use_w:
                        r = smeta[i * 2] & 15
                        wv = wtab[pl.ds(r * DIM + c * 16, 16)]
                        return (v * row) * wv
                    return v * row

                def mb_body(m, _):
                    i0 = i_lo + m * 8
                    contribs = []
                    segs8 = []
                    for u in range(8):
                        i = i0 + u
                        v = jax.lax.bitcast_convert_type(smeta[i * 2 + 1],
                                                         jnp.float32)
                        segs8.append((smeta[i * 2] >> 4) - base)
                        contribs.append(
                            [scaled_row(i, c, v) for c in range(DIM // 16)])
                    for u in range(8):
                        s = segs8[u]
                        for c in range(DIM // 16):
                            acc[s, pl.ds(c * 16, 16)] = (
                                acc[s, pl.ds(c * 16, 16)] + contribs[u][c])
                    return 0

                nmb = lax.max((i_hi - i_lo) // 8, 0)
                lax.fori_loop(0, nmb, mb_body, 0)

                def edge_body(i, _):
                    v = jax.lax.bitcast_convert_type(smeta[i * 2 + 1],
                                                     jnp.float32)
                    s = (smeta[i * 2] >> 4) - base
                    for c in range(DIM // 16):
                        acc[s, pl.ds(c * 16, 16)] = (
                            acc[s, pl.ds(c * 16, 16)] + scaled_row(i, c, v))
                    return 0

                lax.fori_loop(i_lo + nmb * 8, i_hi, edge_body, 0)
                return 0

            lax.fori_loop(0, nch, chunk_body, 0)
            pltpu.sync_copy(acc, out_h.at[pl.ds(base, _R)])

        return 0

    lax.fori_loop(0, _ceil_div(nb, NW), block_body, 0)


def _seg_sum_sc(table, idx, seg, val, n_out, rel=None, wtab=None):
    e = idx.shape[0]
    nb = _ceil_div(n_out, _R)
    e_pad = _ceil_div(e, 128) * 128 + _C
    pad = e_pad - e

    idx_p = jnp.concatenate([idx.astype(jnp.int32),
                             jnp.zeros((pad,), jnp.int32)])
    use_w = wtab is not None
    if use_w:
        combo = seg.astype(jnp.int32) * 16 + rel.astype(jnp.int32)
        w_flat = jnp.concatenate(
            [wtab.reshape(-1), jnp.zeros((1024 - wtab.size,), jnp.float32)])
    else:
        combo = seg.astype(jnp.int32) * 16
        w_flat = jnp.zeros((8,), jnp.float32)
    meta = jnp.stack(
        [combo, jax.lax.bitcast_convert_type(val, jnp.int32)],
        axis=1).reshape(-1)
    meta_p = jnp.concatenate([meta, jnp.zeros((pad * 2,), jnp.int32)])

    boff = jnp.searchsorted(
        seg.astype(jnp.int32),
        (jnp.arange(nb + 1, dtype=jnp.int32) * _R)).astype(jnp.int32)
    boff2d = jnp.concatenate(
        [jnp.stack([boff[:-1], boff[1:]], axis=1),
         jnp.zeros((nb, 14), jnp.int32)], axis=1)

    mesh = plsc.VectorSubcoreMesh(core_axis_name="c", subcore_axis_name="s")
    f = pl.kernel(
        functools.partial(_seg_sum_body, nb, use_w),
        mesh=mesh,
        compiler_params=pltpu.CompilerParams(use_tc_tiling_on_sc=False),
        out_type=jax.ShapeDtypeStruct((nb * _R, DIM), jnp.float32),
        scratch_types=[
            pltpu.VMEM((_R, DIM), jnp.float32),
            pltpu.VMEM((_C,), jnp.int32),
            pltpu.VMEM((_C, DIM), jnp.float32),
            pltpu.SMEM((_C * 2,), jnp.int32),
            pltpu.VMEM((16,), jnp.int32),
            pltpu.VMEM((1024,), jnp.float32),
            pltpu.VMEM_SHARED((NS, _C * 2), jnp.int32),
            pltpu.SemaphoreType.DMA,
        ],
    )
    return f(table, idx_p, meta_p, boff2d, w_flat)


_FBLK = 2048


def _fnorm_body(x_ref, o_ref):
    x = x_ref[...]
    n = jnp.sqrt(jnp.sum(x * x, axis=1, keepdims=True))
    o_ref[...] = x / jnp.maximum(n, 1e-12)


def _fnorm_tc(x):
    n = x.shape[0]
    nb = _ceil_div(n, _FBLK)
    n_pad = nb * _FBLK
    if n_pad != n:
        x = jnp.concatenate([x, jnp.zeros((n_pad - n, DIM), x.dtype)])
    out = pl.pallas_call(
        _fnorm_body,
        grid=(nb,),
        in_specs=[pl.BlockSpec((_FBLK, DIM), lambda i: (i, 0))],
        out_specs=pl.BlockSpec((_FBLK, DIM), lambda i: (i, 0)),
        out_shape=jax.ShapeDtypeStruct((n_pad, DIM), jnp.float32),
    )(x)
    return out


_BM = 512


def _loss_body(u_kg_blk, u_ii_blk, pos_blk, neg_blk, in_all,
               ssl_ref, mf_ref, reg_ref):
    i = pl.program_id(0)

    u_kg = u_kg_blk[...]
    u_ii = u_ii_blk[...]
    u_e = u_kg + u_ii
    pos_e = pos_blk[...]
    neg_e = neg_blk[...]

    def l2n(x):
        return x / jnp.sqrt(jnp.sum(x * x, -1, keepdims=True) + 1e-24)

    inn = in_all[...]

    kn_blk = l2n(u_kg)
    scores = jnp.dot(kn_blk, inn.T, preferred_element_type=jnp.float32) * SCALE
    m = jnp.max(scores, axis=1, keepdims=True)
    lse = jnp.log(jnp.sum(jnp.exp(scores - m), axis=1, keepdims=True)) + m
    in_blk = l2n(u_ii)
    diag = jnp.sum(kn_blk * in_blk, axis=1, keepdims=True) * SCALE
    ssl_part = jnp.sum(lse - diag)

    d = jnp.sum(u_e * (pos_e - neg_e), axis=1, keepdims=True)
    mf_part = jnp.sum(jnp.maximum(-d, 0.0) + jnp.log1p(jnp.exp(-jnp.abs(d))))

    reg_part = (jnp.sum(u_e * u_e) + jnp.sum(pos_e * pos_e) +
                jnp.sum(neg_e * neg_e))

    @pl.when(i == 0)
    def _init():
        ssl_ref[0, 0] = 0.0
        mf_ref[0, 0] = 0.0
        reg_ref[0, 0] = 0.0

    ssl_ref[0, 0] += ssl_part
    mf_ref[0, 0] += mf_part
    reg_ref[0, 0] += reg_part


def _loss_head(u_kg_b, u_ii_b, pos_e, neg_e):
    def l2n(x):
        return x / jnp.sqrt(jnp.sum(x * x, -1, keepdims=True) + 1e-24)

    inn = l2n(u_ii_b)
    grid = (BATCH // _BM,)
    blk = pl.BlockSpec((_BM, DIM), lambda i: (i, 0))
    full = pl.BlockSpec((BATCH, DIM), lambda i: (0, 0))
    out_spec = pl.BlockSpec(memory_space=pltpu.SMEM)
    ssl_s, mf_s, reg_s = pl.pallas_call(
        _loss_body,
        grid=grid,
        in_specs=[blk, blk, blk, blk, full],
        out_specs=[out_spec, out_spec, out_spec],
        out_shape=[jax.ShapeDtypeStruct((1, 1), jnp.float32)] * 3,
    )(u_kg_b, u_ii_b, pos_e, neg_e, inn)
    ssl_loss = ssl_s[0, 0] / BATCH
    mf_loss = mf_s[0, 0] / BATCH
    emb_loss = DECAY * (reg_s[0, 0] / 2.0) / BATCH
    total = mf_loss + emb_loss + ALPHA * ssl_loss
    return total, mf_loss, emb_loss


def kernel(all_embed, weight, kg_val, ii_val, ui_val, users, pos_items,
           neg_items, kg_pairs, kg_row, ii_src, ii_dst, ui_user, ui_item):
    user_emb = all_embed[:N_USERS]
    entity_emb = all_embed[N_USERS:]
    item_emb = entity_emb[:N_ITEMS]

    kg_rel = kg_pairs[:, 0].astype(jnp.int32) - 1
    kg_tail = kg_pairs[:, 1].astype(jnp.int32)

    e1_raw = _seg_sum_sc(entity_emb, kg_tail, kg_row, kg_val, N_ENTITIES,
                         rel=kg_rel, wtab=weight)
    e1n = _fnorm_tc(e1_raw[:N_ENTITIES])[:N_ENTITIES]

    e2_raw = _seg_sum_sc(e1n[:N_ENTITIES], kg_tail, kg_row, kg_val, N_ITEMS,
                         rel=kg_rel, wtab=weight)
    e2n = _fnorm_tc(e2_raw[:N_ITEMS])[:N_ITEMS]

    ii_raw = _seg_sum_sc(item_emb, ii_src, ii_dst, ii_val, N_ITEMS)
    iin = _fnorm_tc(ii_raw[:N_ITEMS])[:N_ITEMS]

    item_gcn = (2.0 * item_emb + e1n[:N_ITEMS] + e2n[:N_ITEMS] + 2.0 * iin)

    uiA_raw = _seg_sum_sc(item_emb, ui_item, ui_user, ui_val, N_USERS)
    uiB_raw = _seg_sum_sc(e1n[:N_ENTITIES], ui_item, ui_user, ui_val, N_USERS)
    uiC_raw = _seg_sum_sc(iin, ui_item, ui_user, ui_val, N_USERS)
    uiA = _fnorm_tc(uiA_raw[:N_USERS])[:N_USERS]
    uiB = _fnorm_tc(uiB_raw[:N_USERS])[:N_USERS]
    uiC = _fnorm_tc(uiC_raw[:N_USERS])[:N_USERS]

    user_kg = user_emb + uiA + uiB
    user_ii = user_emb + uiA + uiC

    u_kg_b = user_kg[users]
    u_ii_b = user_ii[users]
    pos_e = item_gcn[pos_items]
    neg_e = item_gcn[neg_items]

    return _loss_head(u_kg_b, u_ii_b, pos_e, neg_e)
